# baseline (speedup 1.0000x reference)
import functools

import jax
import jax.numpy as jnp
from jax import lax
from jax.experimental import pallas as pl
from jax.experimental.pallas import tpu as pltpu
from jax.experimental.pallas import tpu_sc as plsc

BATCH = 16384
EMBED_DIM = 64
NUM_WORKERS = 32
ROWS_PER_W = BATCH // NUM_WORKERS
IDX_MINOR = 128
IDX_ROWS = ROWS_PER_W // IDX_MINOR
GROUPS = ROWS_PER_W // 16


def _sc_body(uids, iids, uemb, iemb, ubias, ibias, gbias, out,
             uidx_v, iidx_v, urows_v, irows_v, ubias_v, ibias_v,
             gb_v, scores_v, scratch_v, sem):
    nc = 2
    wid = lax.axis_index("s") * nc + lax.axis_index("c")
    base = wid * ROWS_PER_W

    pltpu.sync_copy(uids.at[pl.ds(wid * IDX_ROWS, IDX_ROWS)], uidx_v)
    pltpu.sync_copy(iids.at[pl.ds(wid * IDX_ROWS, IDX_ROWS)], iidx_v)
    pltpu.sync_copy(gbias, gb_v)

    copies = []
    for j in range(IDX_ROWS):
        sl = pl.ds(j * IDX_MINOR, IDX_MINOR)
        copies.append(pltpu.async_copy(uemb.at[uidx_v.at[j]], urows_v.at[sl], sem))
        copies.append(pltpu.async_copy(iemb.at[iidx_v.at[j]], irows_v.at[sl], sem))
        copies.append(pltpu.async_copy(ubias.at[uidx_v.at[j]], ubias_v.at[sl], sem))
        copies.append(pltpu.async_copy(ibias.at[iidx_v.at[j]], ibias_v.at[sl], sem))
    for c in copies:
        c.wait()

    lanes = lax.iota(jnp.int32, 16)
    zeros16 = jnp.zeros((16,), jnp.int32)
    gb = gb_v[:]

    def group_body(g, carry):
        row0 = g * 16
        for r in range(16):
            row = row0 + r
            acc = urows_v[row, pl.ds(0, 16)] * irows_v[row, pl.ds(0, 16)]
            for c in range(1, 4):
                acc = acc + (urows_v[row, pl.ds(c * 16, 16)] *
                             irows_v[row, pl.ds(c * 16, 16)])
            scratch_v[pl.ds(r * 16, 16)] = acc
        tot = ubias_v[pl.ds(row0, 16)] + ibias_v[pl.ds(row0, 16)] + gb
        lanes16 = lanes * 16
        for c in range(16):
            tot = tot + plsc.load_gather(scratch_v, [lanes16 + c])
        scores_v[pl.ds(row0, 16)] = tot
        return carry

    lax.fori_loop(0, GROUPS, group_body, 0)
    pltpu.sync_copy(scores_v, out.at[pl.ds(base, ROWS_PER_W)])


def kernel(user_ids, item_ids, user_emb_w, item_emb_w, user_bias_w,
           item_bias_w, global_bias):
    uids = jnp.asarray(user_ids, jnp.int32).reshape(
        NUM_WORKERS * IDX_ROWS, IDX_MINOR)
    iids = jnp.asarray(item_ids, jnp.int32).reshape(
        NUM_WORKERS * IDX_ROWS, IDX_MINOR)
    gb16 = jnp.broadcast_to(global_bias.astype(jnp.float32), (16,))
    ubias1d = user_bias_w.reshape(-1)
    ibias1d = item_bias_w.reshape(-1)

    mesh = plsc.VectorSubcoreMesh(core_axis_name="c", subcore_axis_name="s")
    run = pl.kernel(
        _sc_body,
        mesh=mesh,
        compiler_params=pltpu.CompilerParams(
            needs_layout_passes=False, use_tc_tiling_on_sc=False),
        out_type=jax.ShapeDtypeStruct((BATCH,), jnp.float32),
        scratch_types=[
            pltpu.VMEM((IDX_ROWS, IDX_MINOR), jnp.int32),
            pltpu.VMEM((IDX_ROWS, IDX_MINOR), jnp.int32),
            pltpu.VMEM((ROWS_PER_W, EMBED_DIM), jnp.float32),
            pltpu.VMEM((ROWS_PER_W, EMBED_DIM), jnp.float32),
            pltpu.VMEM((ROWS_PER_W,), jnp.float32),
            pltpu.VMEM((ROWS_PER_W,), jnp.float32),
            pltpu.VMEM((16,), jnp.float32),
            pltpu.VMEM((ROWS_PER_W,), jnp.float32),
            pltpu.VMEM((256,), jnp.float32),
            pltpu.SemaphoreType.DMA,
        ],
    )
    return run(uids, iids, user_emb_w, item_emb_w, ubias1d, ibias1d, gb16)
